# baseline (speedup 1.0000x reference)
import functools

import jax
import jax.numpy as jnp
from jax import lax
from jax.experimental import pallas as pl
from jax.experimental.pallas import tpu as pltpu
from jax.experimental.pallas import tpu_sc as plsc

D = 64
NC = 2
NS = 16
NW = NC * NS
L = 16


@functools.lru_cache(maxsize=None)
def _build_sc_call(B, K):
    BPW = B // NW
    CH = min(256, BPW)
    NCHUNK = BPW // CH
    POSG = min(128, CH)
    GRP = 4
    SLAB = GRP * K
    NSLAB = CH // GRP

    assert B % NW == 0 and BPW % CH == 0 and CH % GRP == 0 and CH % POSG == 0
    assert SLAB <= 128 and SLAB % 8 == 0 and BPW % 8 == 0

    def body(cidx_hbm, tidx_hbm, nidx_hbm, win_hbm, wout_hbm,
             pos_hbm, neg_hbm,
             gidx_c, gidx_t, gidx_n, c_rows, t_rows, n_rows, pos_v, negd_v,
             st, sem):
        wid = lax.axis_index("s") * NC + lax.axis_index("c")
        base = wid * BPW
        lane = lax.iota(jnp.int32, L)

        def load_row(ref, r):
            return [ref[r, pl.ds(j * L, L)] for j in range(D // L)]

        def prod_fold(av, b_ref, br):
            p = av[0] * b_ref[br, pl.ds(0, L)]
            for j in range(1, D // L):
                p += av[j] * b_ref[br, pl.ds(j * L, L)]
            return p

        def reduce_tile(st):
            dots = plsc.load_gather(st, [lane, jnp.zeros((L,), jnp.int32)])
            for l in range(1, L):
                dots += plsc.load_gather(st, [lane, jnp.full((L,), l, jnp.int32)])
            return dots

        def chunk(ci, carry):
            cb = base + ci * CH
            for h in range(CH // POSG):
                pltpu.sync_copy(cidx_hbm.at[pl.ds(cb + h * POSG, POSG)],
                                gidx_c)
                pltpu.async_copy(
                    win_hbm.at[gidx_c],
                    c_rows.at[pl.ds(h * POSG, POSG)], sem).wait()
                pltpu.sync_copy(tidx_hbm.at[pl.ds(cb + h * POSG, POSG)],
                                gidx_t)
                pltpu.async_copy(
                    wout_hbm.at[gidx_t],
                    t_rows.at[pl.ds(h * POSG, POSG)], sem).wait()

            def pos_grp(g, carry):
                for i in range(L):
                    b = g * L + i
                    st[i, :] = prod_fold(load_row(c_rows, b), t_rows, b)
                pos_v[pl.ds(g * L, L)] = reduce_tile(st)
                return carry
            lax.fori_loop(0, CH // L, pos_grp, 0)

            def slab(s, carry):
                pltpu.sync_copy(
                    nidx_hbm.at[pl.ds(cb * K + s * SLAB, SLAB)], gidx_n)
                pltpu.async_copy(wout_hbm.at[gidx_n], n_rows, sem).wait()
                cv = None
                for r in range(SLAB):
                    if r % K == 0:
                        cv = load_row(c_rows, s * GRP + r // K)
                    st[r % L, :] = prod_fold(cv, n_rows, r)
                    if r % L == L - 1:
                        negd_v[pl.ds(s * SLAB + (r // L) * L, L)] = (
                            reduce_tile(st))
                return carry
            lax.fori_loop(0, NSLAB, slab, 0)

            pltpu.sync_copy(pos_v, pos_hbm.at[pl.ds(cb, CH)])
            pltpu.sync_copy(negd_v, neg_hbm.at[pl.ds(cb * K, CH * K)])
            return carry

        lax.fori_loop(0, NCHUNK, chunk, 0)

    mesh = plsc.VectorSubcoreMesh(core_axis_name="c", subcore_axis_name="s",
                                  num_cores=NC, num_subcores=NS)
    return pl.kernel(
        body,
        out_type=(jax.ShapeDtypeStruct((B,), jnp.float32),
                  jax.ShapeDtypeStruct((B * K,), jnp.float32)),
        mesh=mesh,
        compiler_params=pltpu.CompilerParams(needs_layout_passes=False,
                                             use_tc_tiling_on_sc=False),
        scratch_types=[
            pltpu.VMEM((POSG,), jnp.int32),
            pltpu.VMEM((POSG,), jnp.int32),
            pltpu.VMEM((SLAB,), jnp.int32),
            pltpu.VMEM((CH, D), jnp.float32),
            pltpu.VMEM((CH, D), jnp.float32),
            pltpu.VMEM((SLAB, D), jnp.float32),
            pltpu.VMEM((CH,), jnp.float32),
            pltpu.VMEM((CH * K,), jnp.float32),
            pltpu.VMEM((L, L), jnp.float32),
            pltpu.SemaphoreType.DMA,
        ],
    )


def _loss_body(B, pos_ref, neg_ref, out_ref):
    pos_nll = jnp.log(1.0 + jnp.exp(-pos_ref[...]))
    neg_nll = jnp.log(1.0 + jnp.exp(neg_ref[...]))
    out_ref[0, 0] = (jnp.sum(pos_nll) + jnp.sum(neg_nll)) / B


@functools.lru_cache(maxsize=None)
def _build_loss_call(B, K):
    return pl.pallas_call(
        functools.partial(_loss_body, B),
        out_shape=jax.ShapeDtypeStruct((1, 1), jnp.float32),
        out_specs=pl.BlockSpec(memory_space=pltpu.SMEM),
    )


def kernel(center_words, target_words, neg_words, W_in, W_out):
    B, K = neg_words.shape
    c = center_words.astype(jnp.int32)
    t = target_words.astype(jnp.int32)
    n = neg_words.astype(jnp.int32).reshape(-1)
    pos, negd = _build_sc_call(B, K)(c, t, n, W_in, W_out)
    loss = _build_loss_call(B, K)(pos.reshape(B // 128, 128),
                                  negd.reshape(B * K // 128, 128))
    return loss[0, 0]


# speedup vs baseline: 1.1111x; 1.1111x over previous
import functools

import jax
import jax.numpy as jnp
from jax import lax
from jax.experimental import pallas as pl
from jax.experimental.pallas import tpu as pltpu
from jax.experimental.pallas import tpu_sc as plsc

D = 64
NC = 2
NS = 16
NW = NC * NS
L = 16


@functools.lru_cache(maxsize=None)
def _build_sc_call(B, K):
    BPW = B // NW
    CH = min(512, BPW)
    NCHUNK = BPW // CH
    POSG = min(128, CH)
    GRP = 4
    SLAB = GRP * K
    NSLAB = CH // GRP

    assert B % NW == 0 and BPW % CH == 0 and CH % GRP == 0 and CH % POSG == 0
    assert SLAB <= 128 and SLAB % 8 == 0 and BPW % 8 == 0
    assert SLAB % L == 0 and NSLAB % 2 == 0 and CH % L == 0

    def body(cidx_hbm, tidx_hbm, nidx_hbm, win_hbm, wout_hbm,
             pos_hbm, neg_hbm,
             cidx_v, tidx_v, nidx_v, c_rows, t_rows, n_rows0, n_rows1,
             pos_v, negd_v, st, semc, semt, sem0, sem1):
        wid = lax.axis_index("s") * NC + lax.axis_index("c")
        base = wid * BPW
        lane = lax.iota(jnp.int32, L)

        def load_row(ref, r):
            return [ref[r, pl.ds(j * L, L)] for j in range(D // L)]

        def prod_fold(av, b_ref, br):
            p01 = (av[0] * b_ref[br, pl.ds(0, L)]
                   + av[1] * b_ref[br, pl.ds(L, L)])
            p23 = (av[2] * b_ref[br, pl.ds(2 * L, L)]
                   + av[3] * b_ref[br, pl.ds(3 * L, L)])
            return p01 + p23

        def reduce_tile(st_ref):
            dots = plsc.load_gather(st_ref, [lane, jnp.zeros((L,), jnp.int32)])
            for l in range(1, L):
                dots += plsc.load_gather(
                    st_ref, [lane, jnp.full((L,), l, jnp.int32)])
            return dots

        def chunk(ci, carry):
            cb = base + ci * CH
            pltpu.sync_copy(cidx_hbm.at[pl.ds(cb, CH)], cidx_v)
            pltpu.sync_copy(tidx_hbm.at[pl.ds(cb, CH)], tidx_v)
            pltpu.sync_copy(nidx_hbm.at[pl.ds(cb * K, CH * K)], nidx_v)

            c_handles = [pltpu.async_copy(
                win_hbm.at[cidx_v.at[pl.ds(h * POSG, POSG)]],
                c_rows.at[pl.ds(h * POSG, POSG)], semc)
                for h in range(CH // POSG)]
            t_handles = [pltpu.async_copy(
                wout_hbm.at[tidx_v.at[pl.ds(h * POSG, POSG)]],
                t_rows.at[pl.ds(h * POSG, POSG)], semt)
                for h in range(CH // POSG)]

            pltpu.async_copy(wout_hbm.at[nidx_v.at[pl.ds(0, SLAB)]],
                             n_rows0, sem0)
            for h in c_handles:
                h.wait()

            def compute_slab(s, n_ref):
                cv = None
                for r in range(SLAB):
                    if r % K == 0:
                        cv = load_row(c_rows, s * GRP + r // K)
                    st[r % L, :] = prod_fold(cv, n_ref, r)
                    if r % L == L - 1:
                        negd_v[pl.ds(s * SLAB + (r // L) * L, L)] = (
                            reduce_tile(st))

            def pair(p, carry):
                s0 = 2 * p
                pltpu.async_copy(
                    wout_hbm.at[nidx_v.at[pl.ds((s0 + 1) * SLAB, SLAB)]],
                    n_rows1, sem1)
                pltpu.make_async_copy(
                    wout_hbm.at[nidx_v.at[pl.ds(s0 * SLAB, SLAB)]],
                    n_rows0, sem0).wait()
                compute_slab(s0, n_rows0)

                @pl.when(s0 + 2 < NSLAB)
                def _():
                    pltpu.async_copy(
                        wout_hbm.at[nidx_v.at[pl.ds((s0 + 2) * SLAB, SLAB)]],
                        n_rows0, sem0)
                pltpu.make_async_copy(
                    wout_hbm.at[nidx_v.at[pl.ds((s0 + 1) * SLAB, SLAB)]],
                    n_rows1, sem1).wait()
                compute_slab(s0 + 1, n_rows1)
                return carry

            lax.fori_loop(0, NSLAB // 2, pair, 0)

            for h in t_handles:
                h.wait()

            def pos_grp(g, carry):
                for i in range(L):
                    b = g * L + i
                    st[i, :] = prod_fold(load_row(c_rows, b), t_rows, b)
                pos_v[pl.ds(g * L, L)] = reduce_tile(st)
                return carry
            lax.fori_loop(0, CH // L, pos_grp, 0)

            pltpu.sync_copy(pos_v, pos_hbm.at[pl.ds(cb, CH)])
            pltpu.sync_copy(negd_v, neg_hbm.at[pl.ds(cb * K, CH * K)])
            return carry

        lax.fori_loop(0, NCHUNK, chunk, 0)

    mesh = plsc.VectorSubcoreMesh(core_axis_name="c", subcore_axis_name="s",
                                  num_cores=NC, num_subcores=NS)
    return pl.kernel(
        body,
        out_type=(jax.ShapeDtypeStruct((B,), jnp.float32),
                  jax.ShapeDtypeStruct((B * K,), jnp.float32)),
        mesh=mesh,
        compiler_params=pltpu.CompilerParams(needs_layout_passes=False,
                                             use_tc_tiling_on_sc=False),
        scratch_types=[
            pltpu.VMEM((CH,), jnp.int32),
            pltpu.VMEM((CH,), jnp.int32),
            pltpu.VMEM((CH * K,), jnp.int32),
            pltpu.VMEM((CH, D), jnp.float32),
            pltpu.VMEM((CH, D), jnp.float32),
            pltpu.VMEM((SLAB, D), jnp.float32),
            pltpu.VMEM((SLAB, D), jnp.float32),
            pltpu.VMEM((CH,), jnp.float32),
            pltpu.VMEM((CH * K,), jnp.float32),
            pltpu.VMEM((L, L), jnp.float32),
            pltpu.SemaphoreType.DMA,
            pltpu.SemaphoreType.DMA,
            pltpu.SemaphoreType.DMA,
            pltpu.SemaphoreType.DMA,
        ],
    )


def _loss_body(B, pos_ref, neg_ref, out_ref):
    pos_nll = jnp.log(1.0 + jnp.exp(-pos_ref[...]))
    neg_nll = jnp.log(1.0 + jnp.exp(neg_ref[...]))
    out_ref[0, 0] = (jnp.sum(pos_nll) + jnp.sum(neg_nll)) / B


@functools.lru_cache(maxsize=None)
def _build_loss_call(B, K):
    return pl.pallas_call(
        functools.partial(_loss_body, B),
        out_shape=jax.ShapeDtypeStruct((1, 1), jnp.float32),
        out_specs=pl.BlockSpec(memory_space=pltpu.SMEM),
    )


def kernel(center_words, target_words, neg_words, W_in, W_out):
    B, K = neg_words.shape
    c = center_words.astype(jnp.int32)
    t = target_words.astype(jnp.int32)
    n = neg_words.astype(jnp.int32).reshape(-1)
    pos, negd = _build_sc_call(B, K)(c, t, n, W_in, W_out)
    loss = _build_loss_call(B, K)(pos.reshape(B // 128, 128),
                                  negd.reshape(B * K // 128, 128))
    return loss[0, 0]
